# baseline (speedup 1.0000x reference)
import functools

import jax
import jax.numpy as jnp
from jax import lax
from jax.experimental import pallas as pl
from jax.experimental.pallas import tpu as pltpu
from jax.experimental.pallas import tpu_sc as plsc

D = 1024
SCALE = 32.0
LANES = 16

NC = 2
NS = 16
NW = NC * NS

B_TOTAL = 4 * 8192
BPW = B_TOTAL // NW
C = 64
NCHUNK = BPW // C

_mesh = plsc.VectorSubcoreMesh(core_axis_name="c", subcore_axis_name="s")


@functools.partial(
    pl.kernel,
    mesh=_mesh,
    out_type=jax.ShapeDtypeStruct((B_TOTAL, D), jnp.float32),
    scratch_types=[
        pltpu.VMEM((NCHUNK, C), jnp.int32),
        pltpu.VMEM((C, D), jnp.float32),
        pltpu.SemaphoreType.DMA,
    ],
)
def _emb_lookup(x_hbm, table_hbm, out_hbm, idx_v, rows_v, sem):
    wid = lax.axis_index("s") * NC + lax.axis_index("c")
    base = wid * BPW
    pltpu.sync_copy(x_hbm.at[wid], idx_v)

    def chunk(g, carry):
        pltpu.async_copy(table_hbm.at[idx_v.at[g]], rows_v, sem).wait()

        def row(r, c2):
            def col(c, c3):
                sl = pl.ds(c * LANES, LANES)
                rows_v[r, sl] = rows_v[r, sl] * SCALE
                return c3
            return lax.fori_loop(0, D // LANES, col, c2)

        lax.fori_loop(0, C, row, 0)
        pltpu.sync_copy(rows_v, out_hbm.at[pl.ds(base + g * C, C)])
        return carry

    lax.fori_loop(0, NCHUNK, chunk, 0)


def kernel(x, table):
    xf = x.reshape(NW, NCHUNK, C)
    out = _emb_lookup(xf, table)
    return out.reshape(4, 8192, D)


# speedup vs baseline: 3.7218x; 3.7218x over previous
import functools

import jax
import jax.numpy as jnp
from jax import lax
from jax.experimental import pallas as pl
from jax.experimental.pallas import tpu as pltpu
from jax.experimental.pallas import tpu_sc as plsc

D = 1024
SCALE = 32.0
LANES = 16

NC = 2
NS = 16
NW = NC * NS

B_TOTAL = 4 * 8192
BPW = B_TOTAL // NW
C = 32
NCHUNK = BPW // C
NBUF = 3

_mesh = plsc.VectorSubcoreMesh(core_axis_name="c", subcore_axis_name="s")


@functools.partial(
    pl.kernel,
    mesh=_mesh,
    out_type=jax.ShapeDtypeStruct((B_TOTAL, D), jnp.float32),
    scratch_types=[
        pltpu.VMEM((NCHUNK, C), jnp.int32),
        pltpu.VMEM((NBUF, C, D), jnp.float32),
        pltpu.SemaphoreType.DMA,
        pltpu.SemaphoreType.DMA,
    ],
)
def _emb_lookup(x_hbm, table_hbm, out_hbm, idx_v, rows_v, semg, sems):
    wid = lax.axis_index("s") * NC + lax.axis_index("c")
    base = wid * BPW
    pltpu.sync_copy(x_hbm.at[wid], idx_v)

    def gather(g):
        return pltpu.async_copy(
            table_hbm.at[idx_v.at[g]], rows_v.at[g % NBUF], semg)

    gath = {0: gather(0), 1: gather(1)}
    scat = {}
    pending = []
    for g in range(NCHUNK):
        b = g % NBUF
        gath[g].wait()

        def row(r, carry, b=b):
            for c in range(D // LANES):
                sl = pl.ds(c * LANES, LANES)
                rows_v[b, r, sl] = rows_v[b, r, sl] * SCALE
            return carry

        lax.fori_loop(0, C, row, 0)
        scat[g] = pltpu.async_copy(
            rows_v.at[b], out_hbm.at[pl.ds(base + g * C, C)], sems)
        pending.append(g)
        if g + 2 < NCHUNK:
            if g >= 1:
                scat[g - 1].wait()
                pending.remove(g - 1)
            gath[g + 2] = gather(g + 2)
    for g in pending:
        scat[g].wait()


def kernel(x, table):
    xf = x.reshape(NW, NCHUNK, C)
    out = _emb_lookup(xf, table)
    return out.reshape(4, 8192, D)
